# Initial kernel scaffold; baseline (speedup 1.0000x reference)
import jax
import jax.numpy as jnp
from jax.experimental import pallas as pl


def kernel(x, src_index, dst_index, history_counts, W, b):
    raise NotImplementedError("write your pallas kernel here")


# speedup vs baseline: 5.7407x; 5.7407x over previous
import functools

import jax
import jax.numpy as jnp
from jax import lax
from jax.experimental import pallas as pl
from jax.experimental.pallas import tpu as pltpu
from jax.experimental.pallas import tpu_sc as plsc

N_NODES = 100000
N_EVENTS = 1600000

NC = 2
NS = 16
NW = NC * NS
L = 16

WIN = 3200
N_WIN = N_EVENTS // WIN
BASE_WIN = N_WIN // NW
EXTRA = N_WIN - BASE_WIN * NW


def _proj_body(wm_ref, xt_ref, o_ref):
    o_ref[...] = jnp.dot(wm_ref[...], xt_ref[...],
                         preferred_element_type=jnp.float32)


def _project(wm, xt):
    return pl.pallas_call(
        _proj_body,
        out_shape=jax.ShapeDtypeStruct((4, N_NODES), jnp.float32),
    )(wm, xt)


def _sc_body(s0, s1, d0, d1, src_hbm, dst_hbm, hc_hbm, wb_hbm, out_hbm,
             src_v, dst_v, hc_v, s0_v, s1_v, d0_v, d1_v, o_v, wb_v):
    wid = lax.axis_index("s") * NC + lax.axis_index("c")
    nwin = BASE_WIN + jnp.where(wid < EXTRA, 1, 0)

    pltpu.sync_copy(wb_hbm, wb_v)
    w0v = wb_v[pl.ds(0, L)]
    w1v = wb_v[pl.ds(L, L)]
    b0v = wb_v[pl.ds(2 * L, L)]
    b1v = wb_v[pl.ds(3 * L, L)]
    iota2 = lax.iota(jnp.int32, L) * 2

    def window(i, carry):
        win = wid + NW * i
        base = pl.multiple_of(win * WIN, 8)
        pltpu.sync_copy(src_hbm.at[pl.ds(base, WIN)], src_v)
        pltpu.sync_copy(dst_hbm.at[pl.ds(base, WIN)], dst_v)
        pltpu.sync_copy(hc_hbm.at[pl.ds(base, WIN)], hc_v)
        pltpu.sync_copy(s0.at[src_v], s0_v)
        pltpu.sync_copy(s1.at[src_v], s1_v)
        pltpu.sync_copy(d0.at[dst_v], d0_v)
        pltpu.sync_copy(d1.at[dst_v], d1_v)

        def step(k, carry2):
            sl = pl.ds(k * L, L)
            h = hc_v[sl].astype(jnp.float32)
            o0 = s0_v[sl] + d0_v[sl] + h * w0v + b0v
            o1 = s1_v[sl] + d1_v[sl] + h * w1v + b1v
            idx0 = iota2 + (2 * L) * k
            plsc.store_scatter(o_v, [idx0], o0)
            plsc.store_scatter(o_v, [idx0 + 1], o1)
            return carry2

        lax.fori_loop(0, WIN // L, step, 0)
        pltpu.sync_copy(o_v, out_hbm.at[pl.ds(pl.multiple_of(base * 2, 8), WIN * 2)])
        return carry

    lax.fori_loop(0, nwin, window, 0)


@functools.partial(
    pl.kernel,
    mesh=plsc.VectorSubcoreMesh(core_axis_name="c", subcore_axis_name="s"),
    out_type=jax.ShapeDtypeStruct((N_EVENTS * 2,), jnp.float32),
    compiler_params=pltpu.CompilerParams(needs_layout_passes=False),
    scratch_types=[
        pltpu.VMEM((WIN,), jnp.int32),
        pltpu.VMEM((WIN,), jnp.int32),
        pltpu.VMEM((WIN,), jnp.int32),
        pltpu.VMEM((WIN,), jnp.float32),
        pltpu.VMEM((WIN,), jnp.float32),
        pltpu.VMEM((WIN,), jnp.float32),
        pltpu.VMEM((WIN,), jnp.float32),
        pltpu.VMEM((WIN * 2,), jnp.float32),
        pltpu.VMEM((4 * L,), jnp.float32),
    ],
)
def _sc_kernel(*refs):
    _sc_body(*refs)


def kernel(x, src_index, dst_index, history_counts, W, b):
    wm = jnp.concatenate([W[:, 0:4], W[:, 4:8]], axis=0)
    tbl = _project(wm, x.T)
    s0, s1, d0, d1 = tbl[0], tbl[1], tbl[2], tbl[3]
    wb = jnp.concatenate([
        jnp.full((L,), W[0, 8]), jnp.full((L,), W[1, 8]),
        jnp.full((L,), b[0]), jnp.full((L,), b[1]),
    ])
    out_flat = _sc_kernel(s0, s1, d0, d1, src_index, dst_index,
                          history_counts, wb)
    return out_flat.reshape(N_EVENTS, 2)


# speedup vs baseline: 7.2865x; 1.2693x over previous
import functools

import jax
import jax.numpy as jnp
from jax import lax
from jax.experimental import pallas as pl
from jax.experimental.pallas import tpu as pltpu
from jax.experimental.pallas import tpu_sc as plsc

N_NODES = 100000
N_EVENTS = 1600000

NC = 2
NS = 16
NW = NC * NS
L = 16

WIN = 3200
N_WIN = N_EVENTS // WIN
BASE_WIN = N_WIN // NW
EXTRA = N_WIN - BASE_WIN * NW


def _proj_body(wm_ref, xt_ref, o_ref):
    o_ref[...] = jnp.dot(wm_ref[...], xt_ref[...],
                         preferred_element_type=jnp.float32)


def _project(wm, xt):
    return pl.pallas_call(
        _proj_body,
        out_shape=jax.ShapeDtypeStruct((4, N_NODES), jnp.float32),
    )(wm, xt)


def _sc_body(s0, s1, d0, d1, src_hbm, dst_hbm, hc_hbm, wb_hbm, out_hbm,
             src_v, dst_v, hc_v, s0_v, s1_v, d0_v, d1_v, o_v, wb_v):
    wid = lax.axis_index("s") * NC + lax.axis_index("c")
    nwin = BASE_WIN + jnp.where(wid < EXTRA, 1, 0)

    pltpu.sync_copy(wb_hbm, wb_v)
    w0v = wb_v[pl.ds(0, L)]
    w1v = wb_v[pl.ds(L, L)]
    b0v = wb_v[pl.ds(2 * L, L)]
    b1v = wb_v[pl.ds(3 * L, L)]
    iota = lax.iota(jnp.int32, L)
    zeros = iota * 0
    ones = zeros + 1

    def window(i, carry):
        win = wid + NW * i
        base = pl.multiple_of(win * WIN, 8)
        pltpu.sync_copy(src_hbm.at[pl.ds(base, WIN)], src_v)
        pltpu.sync_copy(dst_hbm.at[pl.ds(base, WIN)], dst_v)
        pltpu.sync_copy(hc_hbm.at[pl.ds(base, WIN)], hc_v)
        pltpu.sync_copy(s0.at[src_v], s0_v)
        pltpu.sync_copy(s1.at[src_v], s1_v)
        pltpu.sync_copy(d0.at[dst_v], d0_v)
        pltpu.sync_copy(d1.at[dst_v], d1_v)

        def step(k, carry2):
            sl = pl.ds(k * L, L)
            h = hc_v[sl].astype(jnp.float32)
            o0 = s0_v[sl] + d0_v[sl] + h * w0v + b0v
            o1 = s1_v[sl] + d1_v[sl] + h * w1v + b1v
            e = iota + k * L
            plsc.store_scatter(o_v, [e, zeros], o0)
            plsc.store_scatter(o_v, [e, ones], o1)
            return carry2

        lax.fori_loop(0, WIN // L, step, 0)
        pltpu.sync_copy(o_v, out_hbm.at[pl.ds(base, WIN), :])
        return carry

    lax.fori_loop(0, nwin, window, 0)


@functools.partial(
    pl.kernel,
    mesh=plsc.VectorSubcoreMesh(core_axis_name="c", subcore_axis_name="s"),
    out_type=jax.ShapeDtypeStruct((N_EVENTS, 2), jnp.float32),
    compiler_params=pltpu.CompilerParams(needs_layout_passes=False,
                                         use_tc_tiling_on_sc=False),
    scratch_types=[
        pltpu.VMEM((WIN,), jnp.int32),
        pltpu.VMEM((WIN,), jnp.int32),
        pltpu.VMEM((WIN,), jnp.int32),
        pltpu.VMEM((WIN,), jnp.float32),
        pltpu.VMEM((WIN,), jnp.float32),
        pltpu.VMEM((WIN,), jnp.float32),
        pltpu.VMEM((WIN,), jnp.float32),
        pltpu.VMEM((WIN, 2), jnp.float32),
        pltpu.VMEM((4 * L,), jnp.float32),
    ],
)
def _sc_kernel(*refs):
    _sc_body(*refs)


def kernel(x, src_index, dst_index, history_counts, W, b):
    wm = jnp.concatenate([W[:, 0:4], W[:, 4:8]], axis=0)
    tbl = _project(wm, x.T)
    wb = jnp.concatenate([
        jnp.full((L,), W[0, 8]), jnp.full((L,), W[1, 8]),
        jnp.full((L,), b[0]), jnp.full((L,), b[1]),
    ])
    return _sc_kernel(tbl[0], tbl[1], tbl[2], tbl[3], src_index, dst_index,
                      history_counts, wb)


# speedup vs baseline: 8.5069x; 1.1675x over previous
import functools

import jax
import jax.numpy as jnp
from jax import lax
from jax.experimental import pallas as pl
from jax.experimental.pallas import tpu as pltpu
from jax.experimental.pallas import tpu_sc as plsc

N_NODES = 100000
N_EVENTS = 1600000

NC = 2
NS = 16
NW = NC * NS
L = 16

NP = 100096
NPW = NP // NS

WIN = 3200
N_WIN = N_EVENTS // WIN
BASE_WIN = N_WIN // NW
EXTRA = N_WIN - BASE_WIN * NW


def _sc_body(xf, src_hbm, dst_hbm, hc_hbm, wb_hbm, out_hbm,
             xv, tv, src_v, dst_v, hc_v, s0_v, s1_v, d0_v, d1_v, o_v, wb_v,
             sh_s0, sh_s1, sh_d0, sh_d1):
    cid = lax.axis_index("c")
    sid = lax.axis_index("s")
    wid = sid * NC + cid
    nwin = BASE_WIN + jnp.where(wid < EXTRA, 1, 0)

    pltpu.sync_copy(wb_hbm, wb_v)
    iota = lax.iota(jnp.int32, L)

    nbase = sid * NPW
    pltpu.sync_copy(xf.at[pl.ds(nbase * 4, NPW * 4)], xv)
    cs = [wb_v[pl.ds((4 + j) * L, L)] for j in range(16)]

    def build(k, carry):
        xb = iota * 4 + (4 * L) * k
        x0 = plsc.load_gather(xv, [xb])
        x1 = plsc.load_gather(xv, [xb + 1])
        x2 = plsc.load_gather(xv, [xb + 2])
        x3 = plsc.load_gather(xv, [xb + 3])
        sl = pl.ds(k * L, L)
        tv.at[0][sl] = x0 * cs[0] + x1 * cs[1] + x2 * cs[2] + x3 * cs[3]
        tv.at[1][sl] = x0 * cs[4] + x1 * cs[5] + x2 * cs[6] + x3 * cs[7]
        tv.at[2][sl] = x0 * cs[8] + x1 * cs[9] + x2 * cs[10] + x3 * cs[11]
        tv.at[3][sl] = x0 * cs[12] + x1 * cs[13] + x2 * cs[14] + x3 * cs[15]
        return carry

    lax.fori_loop(0, NPW // L, build, 0)
    pltpu.sync_copy(tv.at[0], sh_s0.at[pl.ds(nbase, NPW)])
    pltpu.sync_copy(tv.at[1], sh_s1.at[pl.ds(nbase, NPW)])
    pltpu.sync_copy(tv.at[2], sh_d0.at[pl.ds(nbase, NPW)])
    pltpu.sync_copy(tv.at[3], sh_d1.at[pl.ds(nbase, NPW)])
    plsc.subcore_barrier()

    w0v = wb_v[pl.ds(0, L)]
    w1v = wb_v[pl.ds(L, L)]
    b0v = wb_v[pl.ds(2 * L, L)]
    b1v = wb_v[pl.ds(3 * L, L)]
    zeros = iota * 0
    ones = zeros + 1

    def window(i, carry):
        win = wid + NW * i
        base = pl.multiple_of(win * WIN, 8)
        pltpu.sync_copy(src_hbm.at[pl.ds(base, WIN)], src_v)
        pltpu.sync_copy(dst_hbm.at[pl.ds(base, WIN)], dst_v)
        pltpu.sync_copy(hc_hbm.at[pl.ds(base, WIN)], hc_v)
        pltpu.sync_copy(sh_s0.at[src_v], s0_v)
        pltpu.sync_copy(sh_s1.at[src_v], s1_v)
        pltpu.sync_copy(sh_d0.at[dst_v], d0_v)
        pltpu.sync_copy(sh_d1.at[dst_v], d1_v)

        def step(k, carry2):
            sl = pl.ds(k * L, L)
            h = hc_v[sl].astype(jnp.float32)
            o0 = s0_v[sl] + d0_v[sl] + h * w0v + b0v
            o1 = s1_v[sl] + d1_v[sl] + h * w1v + b1v
            e = iota + k * L
            plsc.store_scatter(o_v, [e, zeros], o0)
            plsc.store_scatter(o_v, [e, ones], o1)
            return carry2

        lax.fori_loop(0, WIN // L, step, 0)
        pltpu.sync_copy(o_v, out_hbm.at[pl.ds(base, WIN), :])
        return carry

    lax.fori_loop(0, nwin, window, 0)


@functools.partial(
    pl.kernel,
    mesh=plsc.VectorSubcoreMesh(core_axis_name="c", subcore_axis_name="s"),
    out_type=jax.ShapeDtypeStruct((N_EVENTS, 2), jnp.float32),
    compiler_params=pltpu.CompilerParams(needs_layout_passes=False,
                                         use_tc_tiling_on_sc=False),
    scratch_types=[
        pltpu.VMEM((NPW * 4,), jnp.float32),
        pltpu.VMEM((4, NPW), jnp.float32),
        pltpu.VMEM((WIN,), jnp.int32),
        pltpu.VMEM((WIN,), jnp.int32),
        pltpu.VMEM((WIN,), jnp.int32),
        pltpu.VMEM((WIN,), jnp.float32),
        pltpu.VMEM((WIN,), jnp.float32),
        pltpu.VMEM((WIN,), jnp.float32),
        pltpu.VMEM((WIN,), jnp.float32),
        pltpu.VMEM((WIN, 2), jnp.float32),
        pltpu.VMEM((20 * L,), jnp.float32),
        pltpu.VMEM_SHARED((NP,), jnp.float32),
        pltpu.VMEM_SHARED((NP,), jnp.float32),
        pltpu.VMEM_SHARED((NP,), jnp.float32),
        pltpu.VMEM_SHARED((NP,), jnp.float32),
    ],
)
def _sc_kernel(*refs):
    _sc_body(*refs)


def kernel(x, src_index, dst_index, history_counts, W, b):
    xf = jnp.concatenate([x.reshape(N_NODES * 4),
                          jnp.zeros((NP - N_NODES) * 4, jnp.float32)])
    coeffs = jnp.concatenate([
        W[:, 8], b,
        W[0, 0:4], W[1, 0:4],
        W[0, 4:8], W[1, 4:8],
    ])
    wb = jnp.repeat(coeffs, L)
    return _sc_kernel(xf, src_index, dst_index, history_counts, wb)


# speedup vs baseline: 36.4021x; 4.2791x over previous
import functools

import jax
import jax.numpy as jnp
from jax import lax
from jax.experimental import pallas as pl
from jax.experimental.pallas import tpu as pltpu
from jax.experimental.pallas import tpu_sc as plsc

N_NODES = 100000
N_EVENTS = 1600000

NC = 2
NS = 16
NW = NC * NS
L = 16

NP = 100096
NPW = NP // NS

WIN = 3200
N_WIN = N_EVENTS // WIN
BASE_WIN = N_WIN // NW
EXTRA = N_WIN - BASE_WIN * NW


def _sc_body(xf, src_hbm, dst_hbm, hc_hbm, wb_hbm, out_hbm,
             xv, tv, src_v, dst_v, hc_v, s0_v, s1_v, d0_v, d1_v, o_v, wb_v,
             sh_s0, sh_s1, sh_d0, sh_d1):
    cid = lax.axis_index("c")
    sid = lax.axis_index("s")
    wid = sid * NC + cid
    nwin = BASE_WIN + jnp.where(wid < EXTRA, 1, 0)

    pltpu.sync_copy(wb_hbm, wb_v)
    iota = lax.iota(jnp.int32, L)

    nbase = sid * NPW
    pltpu.sync_copy(xf.at[pl.ds(nbase * 4, NPW * 4)], xv)
    cs = [wb_v[pl.ds((4 + j) * L, L)] for j in range(16)]

    def build(k, carry):
        xb = iota * 4 + (4 * L) * k
        x0 = plsc.load_gather(xv, [xb])
        x1 = plsc.load_gather(xv, [xb + 1])
        x2 = plsc.load_gather(xv, [xb + 2])
        x3 = plsc.load_gather(xv, [xb + 3])
        sl = pl.ds(k * L, L)
        tv.at[0][sl] = x0 * cs[0] + x1 * cs[1] + x2 * cs[2] + x3 * cs[3]
        tv.at[1][sl] = x0 * cs[4] + x1 * cs[5] + x2 * cs[6] + x3 * cs[7]
        tv.at[2][sl] = x0 * cs[8] + x1 * cs[9] + x2 * cs[10] + x3 * cs[11]
        tv.at[3][sl] = x0 * cs[12] + x1 * cs[13] + x2 * cs[14] + x3 * cs[15]
        return carry

    lax.fori_loop(0, NPW // L, build, 0)
    pltpu.sync_copy(tv.at[0], sh_s0.at[pl.ds(nbase, NPW)])
    pltpu.sync_copy(tv.at[1], sh_s1.at[pl.ds(nbase, NPW)])
    pltpu.sync_copy(tv.at[2], sh_d0.at[pl.ds(nbase, NPW)])
    pltpu.sync_copy(tv.at[3], sh_d1.at[pl.ds(nbase, NPW)])
    plsc.subcore_barrier()

    w0v = wb_v[pl.ds(0, L)]
    w1v = wb_v[pl.ds(L, L)]
    b0v = wb_v[pl.ds(2 * L, L)]
    b1v = wb_v[pl.ds(3 * L, L)]

    def window(i, carry):
        win = wid + NW * i
        base = pl.multiple_of(win * WIN, 8)
        pltpu.sync_copy(src_hbm.at[pl.ds(base, WIN)], src_v)
        pltpu.sync_copy(dst_hbm.at[pl.ds(base, WIN)], dst_v)
        pltpu.sync_copy(hc_hbm.at[pl.ds(base, WIN)], hc_v)
        pltpu.sync_copy(sh_s0.at[src_v], s0_v)
        pltpu.sync_copy(sh_s1.at[src_v], s1_v)
        pltpu.sync_copy(sh_d0.at[dst_v], d0_v)
        pltpu.sync_copy(sh_d1.at[dst_v], d1_v)

        def step(k, carry2):
            sl = pl.ds(k * L, L)
            h = hc_v[sl].astype(jnp.float32)
            o0 = s0_v[sl] + d0_v[sl] + h * w0v + b0v
            o1 = s1_v[sl] + d1_v[sl] + h * w1v + b1v
            blk = k >> 3
            sub = pl.ds((k & 7) * L, L)
            o_v[blk, 0, sub] = o0
            o_v[blk, 1, sub] = o1
            return carry2

        lax.fori_loop(0, WIN // L, step, 0)
        pltpu.sync_copy(o_v, out_hbm.at[pl.ds(win * (WIN // 128), WIN // 128), :, :])
        return carry

    lax.fori_loop(0, nwin, window, 0)


@functools.partial(
    pl.kernel,
    mesh=plsc.VectorSubcoreMesh(core_axis_name="c", subcore_axis_name="s"),
    out_type=jax.ShapeDtypeStruct((N_EVENTS // 128, 2, 128), jnp.float32),
    compiler_params=pltpu.CompilerParams(needs_layout_passes=False,
                                         use_tc_tiling_on_sc=False),
    scratch_types=[
        pltpu.VMEM((NPW * 4,), jnp.float32),
        pltpu.VMEM((4, NPW), jnp.float32),
        pltpu.VMEM((WIN,), jnp.int32),
        pltpu.VMEM((WIN,), jnp.int32),
        pltpu.VMEM((WIN,), jnp.int32),
        pltpu.VMEM((WIN,), jnp.float32),
        pltpu.VMEM((WIN,), jnp.float32),
        pltpu.VMEM((WIN,), jnp.float32),
        pltpu.VMEM((WIN,), jnp.float32),
        pltpu.VMEM((WIN // 128, 2, 128), jnp.float32),
        pltpu.VMEM((20 * L,), jnp.float32),
        pltpu.VMEM_SHARED((NP,), jnp.float32),
        pltpu.VMEM_SHARED((NP,), jnp.float32),
        pltpu.VMEM_SHARED((NP,), jnp.float32),
        pltpu.VMEM_SHARED((NP,), jnp.float32),
    ],
)
def _sc_kernel(*refs):
    _sc_body(*refs)


def kernel(x, src_index, dst_index, history_counts, W, b):
    xf = jnp.concatenate([x.reshape(N_NODES * 4),
                          jnp.zeros((NP - N_NODES) * 4, jnp.float32)])
    coeffs = jnp.concatenate([
        W[:, 8], b,
        W[0, 0:4], W[1, 0:4],
        W[0, 4:8], W[1, 4:8],
    ])
    wb = jnp.repeat(coeffs, L)
    o3 = _sc_kernel(xf, src_index, dst_index, history_counts, wb)
    return o3.swapaxes(1, 2).reshape(N_EVENTS, 2)


# speedup vs baseline: 48.9577x; 1.3449x over previous
import functools

import jax
import jax.numpy as jnp
from jax import lax
from jax.experimental import pallas as pl
from jax.experimental.pallas import tpu as pltpu
from jax.experimental.pallas import tpu_sc as plsc

N_NODES = 100000
N_EVENTS = 1600000

NC = 2
NS = 16
NW = NC * NS
L = 16

NP = 102400
NPW = NP // NS
BCH = 1600
NBCH = NPW // BCH

WIN = 3200
N_WIN = N_EVENTS // WIN
NWIN_MAX = -(-N_WIN // NW)
EXTRA = N_WIN - (NWIN_MAX - 1) * NW


def _sc_body(xf, src_hbm, dst_hbm, hc_hbm, wb_hbm, out_hbm,
             xv, tv, src_v, dst_v, hc_v, gv, o_v, wb_v,
             sh_s0, sh_s1, sh_d0, sh_d1, gsem0, gsem1, osem0, osem1):
    cid = lax.axis_index("c")
    sid = lax.axis_index("s")
    wid = sid * NC + cid

    pltpu.sync_copy(wb_hbm, wb_v)
    iota = lax.iota(jnp.int32, L)

    nbase = sid * NPW
    cs = [wb_v[pl.ds((4 + j) * L, L)] for j in range(16)]
    sh = [sh_s0, sh_s1, sh_d0, sh_d1]

    def build_chunk(c, carry):
        cbase = nbase + c * BCH

        def build(k, carry2):
            xb = iota * 4 + (4 * L) * k
            x0 = plsc.load_gather(xv, [xb])
            x1 = plsc.load_gather(xv, [xb + 1])
            x2 = plsc.load_gather(xv, [xb + 2])
            x3 = plsc.load_gather(xv, [xb + 3])
            sl = pl.ds(k * L, L)
            tv.at[0][sl] = x0 * cs[0] + x1 * cs[1] + x2 * cs[2] + x3 * cs[3]
            tv.at[1][sl] = x0 * cs[4] + x1 * cs[5] + x2 * cs[6] + x3 * cs[7]
            tv.at[2][sl] = x0 * cs[8] + x1 * cs[9] + x2 * cs[10] + x3 * cs[11]
            tv.at[3][sl] = (x0 * cs[12] + x1 * cs[13] + x2 * cs[14]
                            + x3 * cs[15])
            return carry2

        pltpu.sync_copy(xf.at[pl.ds(cbase * 4, BCH * 4)], xv)
        lax.fori_loop(0, BCH // L, build, 0)
        for t in range(4):
            pltpu.sync_copy(tv.at[t], sh[t].at[pl.ds(cbase, BCH)])
        return carry

    lax.fori_loop(0, NBCH, build_chunk, 0)
    plsc.subcore_barrier()

    w0v = wb_v[pl.ds(0, L)]
    w1v = wb_v[pl.ds(L, L)]
    b0v = wb_v[pl.ds(2 * L, L)]
    b1v = wb_v[pl.ds(3 * L, L)]
    gsem = [gsem0, gsem1]
    osem = [osem0, osem1]
    tail = wid < EXTRA

    def idx_load(p, i):
        base = pl.multiple_of((wid + NW * i) * WIN, 8)
        pltpu.sync_copy(src_hbm.at[pl.ds(base, WIN)], src_v.at[p])
        pltpu.sync_copy(dst_hbm.at[pl.ds(base, WIN)], dst_v.at[p])
        pltpu.sync_copy(hc_hbm.at[pl.ds(base, WIN)], hc_v.at[p])

    def gathers(p):
        return [
            pltpu.make_async_copy(sh_s0.at[src_v.at[p]], gv.at[p, 0], gsem[p]),
            pltpu.make_async_copy(sh_s1.at[src_v.at[p]], gv.at[p, 1], gsem[p]),
            pltpu.make_async_copy(sh_d0.at[dst_v.at[p]], gv.at[p, 2], gsem[p]),
            pltpu.make_async_copy(sh_d1.at[dst_v.at[p]], gv.at[p, 3], gsem[p]),
        ]

    def out_copy(p, i):
        blk = (wid + NW * i) * (WIN // 128)
        return pltpu.make_async_copy(
            o_v.at[p], out_hbm.at[pl.ds(blk, WIN // 128)], osem[p])

    def compute(p):
        def step(k, carry2):
            sl = pl.ds(k * L, L)
            h = hc_v.at[p][sl].astype(jnp.float32)
            o0 = gv.at[p, 0][sl] + gv.at[p, 2][sl] + h * w0v + b0v
            o1 = gv.at[p, 1][sl] + gv.at[p, 3][sl] + h * w1v + b1v
            blk = k >> 3
            sub = pl.ds((k & 7) * L, L)
            o_v[p, blk, 0, sub] = o0
            o_v[p, blk, 1, sub] = o1
            return carry2

        lax.fori_loop(0, WIN // L, step, 0)

    def run_window(p, i):
        for g in gathers(p):
            g.wait()
        if i >= 2:
            out_copy(p, i - 2).wait()
        compute(p)
        out_copy(p, i).start()

    idx_load(0, 0)
    for g in gathers(0):
        g.start()

    for i in range(NWIN_MAX):
        p = i % 2
        ni = i + 1
        if ni < NWIN_MAX - 1:
            idx_load(ni % 2, ni)
            for g in gathers(ni % 2):
                g.start()
        elif ni == NWIN_MAX - 1:
            @pl.when(tail)
            def _():
                idx_load(ni % 2, ni)
                for g in gathers(ni % 2):
                    g.start()

        if i < NWIN_MAX - 1:
            run_window(p, i)
        else:
            @pl.when(tail)
            def _():
                run_window(p, i)

    @pl.when(tail)
    def _():
        out_copy(1, NWIN_MAX - 1).wait()

    @pl.when(jnp.logical_not(tail))
    def _():
        out_copy(1, NWIN_MAX - 3).wait()

    out_copy(0, NWIN_MAX - 2).wait()


@functools.partial(
    pl.kernel,
    mesh=plsc.VectorSubcoreMesh(core_axis_name="c", subcore_axis_name="s"),
    out_type=jax.ShapeDtypeStruct((N_EVENTS // 128, 2, 128), jnp.float32),
    compiler_params=pltpu.CompilerParams(needs_layout_passes=False,
                                         use_tc_tiling_on_sc=False),
    scratch_types=[
        pltpu.VMEM((BCH * 4,), jnp.float32),
        pltpu.VMEM((4, BCH), jnp.float32),
        pltpu.VMEM((2, WIN), jnp.int32),
        pltpu.VMEM((2, WIN), jnp.int32),
        pltpu.VMEM((2, WIN), jnp.int32),
        pltpu.VMEM((2, 4, WIN), jnp.float32),
        pltpu.VMEM((2, WIN // 128, 2, 128), jnp.float32),
        pltpu.VMEM((20 * L,), jnp.float32),
        pltpu.VMEM_SHARED((NP,), jnp.float32),
        pltpu.VMEM_SHARED((NP,), jnp.float32),
        pltpu.VMEM_SHARED((NP,), jnp.float32),
        pltpu.VMEM_SHARED((NP,), jnp.float32),
        pltpu.SemaphoreType.DMA,
        pltpu.SemaphoreType.DMA,
        pltpu.SemaphoreType.DMA,
        pltpu.SemaphoreType.DMA,
    ],
)
def _sc_kernel(*refs):
    _sc_body(*refs)


def kernel(x, src_index, dst_index, history_counts, W, b):
    xf = jnp.concatenate([x.reshape(N_NODES * 4),
                          jnp.zeros((NP - N_NODES) * 4, jnp.float32)])
    coeffs = jnp.concatenate([
        W[:, 8], b,
        W[0, 0:4], W[1, 0:4],
        W[0, 4:8], W[1, 4:8],
    ])
    wb = jnp.repeat(coeffs, L)
    o3 = _sc_kernel(xf, src_index, dst_index, history_counts, wb)
    return o3.swapaxes(1, 2).reshape(N_EVENTS, 2)


# speedup vs baseline: 51.0141x; 1.0420x over previous
import functools

import jax
import jax.numpy as jnp
from jax import lax
from jax.experimental import pallas as pl
from jax.experimental.pallas import tpu as pltpu
from jax.experimental.pallas import tpu_sc as plsc

N_NODES = 100000
N_EVENTS = 1600000

NC = 2
NS = 16
NW = NC * NS
L = 16

NP = 102400
NPW = NP // NS
BCH = 1600
NBCH = NPW // BCH

WIN = 3200
N_WIN = N_EVENTS // WIN
NWIN_MAX = -(-N_WIN // NW)
EXTRA = N_WIN - (NWIN_MAX - 1) * NW


def _sc_body(xf, src_hbm, dst_hbm, hc_hbm, wb_hbm, out_hbm,
             xv, tv, src_v, dst_v, hc_v, gv, o_v, wb_v,
             sh_s, sh_d, gsem0, gsem1, osem0, osem1):
    cid = lax.axis_index("c")
    sid = lax.axis_index("s")
    wid = sid * NC + cid

    pltpu.sync_copy(wb_hbm, wb_v)
    iota = lax.iota(jnp.int32, L)

    nbase = sid * NPW
    cs = [wb_v[pl.ds((4 + j) * L, L)] for j in range(16)]
    rnd = jnp.int32(0x8000)
    himask = jnp.int32(-65536)

    def pack_bf16(lo, hi):
        ulo = plsc.bitcast(lo, jnp.int32)
        uhi = plsc.bitcast(hi, jnp.int32)
        low = ((ulo + rnd) >> 16) & jnp.int32(0xFFFF)
        high = (uhi + rnd) & himask
        return low | high

    def build_chunk(c, carry):
        cbase = nbase + c * BCH

        def build(k, carry2):
            xb = iota * 4 + (4 * L) * k
            x0 = plsc.load_gather(xv, [xb])
            x1 = plsc.load_gather(xv, [xb + 1])
            x2 = plsc.load_gather(xv, [xb + 2])
            x3 = plsc.load_gather(xv, [xb + 3])
            sl = pl.ds(k * L, L)
            s0 = x0 * cs[0] + x1 * cs[1] + x2 * cs[2] + x3 * cs[3]
            s1 = x0 * cs[4] + x1 * cs[5] + x2 * cs[6] + x3 * cs[7]
            d0 = x0 * cs[8] + x1 * cs[9] + x2 * cs[10] + x3 * cs[11]
            d1 = x0 * cs[12] + x1 * cs[13] + x2 * cs[14] + x3 * cs[15]
            tv.at[0][sl] = pack_bf16(s0, s1)
            tv.at[1][sl] = pack_bf16(d0, d1)
            return carry2

        pltpu.sync_copy(xf.at[pl.ds(cbase * 4, BCH * 4)], xv)
        lax.fori_loop(0, BCH // L, build, 0)
        pltpu.sync_copy(tv.at[0], sh_s.at[pl.ds(cbase, BCH)])
        pltpu.sync_copy(tv.at[1], sh_d.at[pl.ds(cbase, BCH)])
        return carry

    lax.fori_loop(0, NBCH, build_chunk, 0)
    plsc.subcore_barrier()

    w0v = wb_v[pl.ds(0, L)]
    w1v = wb_v[pl.ds(L, L)]
    b0v = wb_v[pl.ds(2 * L, L)]
    b1v = wb_v[pl.ds(3 * L, L)]
    gsem = [gsem0, gsem1]
    osem = [osem0, osem1]
    tail = wid < EXTRA

    def idx_load(p, i):
        base = pl.multiple_of((wid + NW * i) * WIN, 8)
        pltpu.sync_copy(src_hbm.at[pl.ds(base, WIN)], src_v.at[p])
        pltpu.sync_copy(dst_hbm.at[pl.ds(base, WIN)], dst_v.at[p])
        pltpu.sync_copy(hc_hbm.at[pl.ds(base, WIN)], hc_v.at[p])

    def gathers(p):
        return [
            pltpu.make_async_copy(sh_s.at[src_v.at[p]], gv.at[p, 0], gsem[p]),
            pltpu.make_async_copy(sh_d.at[dst_v.at[p]], gv.at[p, 1], gsem[p]),
        ]

    def out_copy(p, i):
        blk = (wid + NW * i) * (WIN // 128)
        return pltpu.make_async_copy(
            o_v.at[p], out_hbm.at[pl.ds(blk, WIN // 128)], osem[p])

    def compute(p):
        def step(k, carry2):
            sl = pl.ds(k * L, L)
            h = hc_v.at[p][sl].astype(jnp.float32)
            gs = gv.at[p, 0][sl]
            gd = gv.at[p, 1][sl]
            s0 = plsc.bitcast(gs << 16, jnp.float32)
            s1 = plsc.bitcast(gs & himask, jnp.float32)
            d0 = plsc.bitcast(gd << 16, jnp.float32)
            d1 = plsc.bitcast(gd & himask, jnp.float32)
            o0 = s0 + d0 + h * w0v + b0v
            o1 = s1 + d1 + h * w1v + b1v
            blk = k >> 3
            sub = pl.ds((k & 7) * L, L)
            o_v[p, blk, 0, sub] = o0
            o_v[p, blk, 1, sub] = o1
            return carry2

        lax.fori_loop(0, WIN // L, step, 0)

    def run_window(p, i):
        for g in gathers(p):
            g.wait()
        if i >= 2:
            out_copy(p, i - 2).wait()
        compute(p)
        out_copy(p, i).start()

    idx_load(0, 0)
    for g in gathers(0):
        g.start()

    for i in range(NWIN_MAX):
        p = i % 2
        ni = i + 1
        if ni < NWIN_MAX - 1:
            idx_load(ni % 2, ni)
            for g in gathers(ni % 2):
                g.start()
        elif ni == NWIN_MAX - 1:
            @pl.when(tail)
            def _():
                idx_load(ni % 2, ni)
                for g in gathers(ni % 2):
                    g.start()

        if i < NWIN_MAX - 1:
            run_window(p, i)
        else:
            @pl.when(tail)
            def _():
                run_window(p, i)

    @pl.when(tail)
    def _():
        out_copy(1, NWIN_MAX - 1).wait()

    @pl.when(jnp.logical_not(tail))
    def _():
        out_copy(1, NWIN_MAX - 3).wait()

    out_copy(0, NWIN_MAX - 2).wait()


@functools.partial(
    pl.kernel,
    mesh=plsc.VectorSubcoreMesh(core_axis_name="c", subcore_axis_name="s"),
    out_type=jax.ShapeDtypeStruct((N_EVENTS // 128, 2, 128), jnp.float32),
    compiler_params=pltpu.CompilerParams(needs_layout_passes=False,
                                         use_tc_tiling_on_sc=False),
    scratch_types=[
        pltpu.VMEM((BCH * 4,), jnp.float32),
        pltpu.VMEM((2, BCH), jnp.int32),
        pltpu.VMEM((2, WIN), jnp.int32),
        pltpu.VMEM((2, WIN), jnp.int32),
        pltpu.VMEM((2, WIN), jnp.int32),
        pltpu.VMEM((2, 2, WIN), jnp.int32),
        pltpu.VMEM((2, WIN // 128, 2, 128), jnp.float32),
        pltpu.VMEM((20 * L,), jnp.float32),
        pltpu.VMEM_SHARED((NP,), jnp.int32),
        pltpu.VMEM_SHARED((NP,), jnp.int32),
        pltpu.SemaphoreType.DMA,
        pltpu.SemaphoreType.DMA,
        pltpu.SemaphoreType.DMA,
        pltpu.SemaphoreType.DMA,
    ],
)
def _sc_kernel(*refs):
    _sc_body(*refs)


def kernel(x, src_index, dst_index, history_counts, W, b):
    xf = jnp.concatenate([x.reshape(N_NODES * 4),
                          jnp.zeros((NP - N_NODES) * 4, jnp.float32)])
    coeffs = jnp.concatenate([
        W[:, 8], b,
        W[0, 0:4], W[1, 0:4],
        W[0, 4:8], W[1, 4:8],
    ])
    wb = jnp.repeat(coeffs, L)
    o3 = _sc_kernel(xf, src_index, dst_index, history_counts, wb)
    return o3.swapaxes(1, 2).reshape(N_EVENTS, 2)


# speedup vs baseline: 51.4871x; 1.0093x over previous
import functools

import jax
import jax.numpy as jnp
from jax import lax
from jax.experimental import pallas as pl
from jax.experimental.pallas import tpu as pltpu
from jax.experimental.pallas import tpu_sc as plsc

N_NODES = 100000
N_EVENTS = 1600000

NC = 2
NS = 16
NW = NC * NS
L = 16

BCH = 2000
NBCH = N_NODES // BCH
BCH_EXTRA = NBCH - (NBCH // NS) * NS

WIN = 3200
N_WIN = N_EVENTS // WIN
NWIN_MAX = -(-N_WIN // NW)
EXTRA = N_WIN - (NWIN_MAX - 1) * NW


def _sc_body(xf, src_hbm, dst_hbm, hc_hbm, wb_hbm, out_hbm,
             xv, tv, src_v, dst_v, hc_v, gv, o_v, wb_v,
             sh_s, sh_d, gsem0, gsem1, osem0, osem1):
    cid = lax.axis_index("c")
    sid = lax.axis_index("s")
    wid = sid * NC + cid

    pltpu.sync_copy(wb_hbm, wb_v)
    iota = lax.iota(jnp.int32, L)

    wb_lo = wb_v[pl.ds(0, L)]
    wb_hi = wb_v[pl.ds(L, L)]

    def splat(j):
        v = wb_lo if j < L else wb_hi
        return jnp.full((L,), v[j % L], jnp.float32)

    cs = ([splat(j) for j in range(4)] + [splat(9 + j) for j in range(4)]
          + [splat(4 + j) for j in range(4)] + [splat(13 + j) for j in range(4)])
    rnd = jnp.int32(0x8000)
    himask = jnp.int32(-65536)

    def pack_bf16(lo, hi):
        ulo = plsc.bitcast(lo, jnp.int32)
        uhi = plsc.bitcast(hi, jnp.int32)
        low = ((ulo + rnd) >> 16) & jnp.int32(0xFFFF)
        high = (uhi + rnd) & himask
        return low | high

    def build_chunk(c, carry):
        cbase = (sid + NS * c) * BCH

        def build(k, carry2):
            xb = iota * 4 + (4 * L) * k
            x0 = plsc.load_gather(xv, [xb])
            x1 = plsc.load_gather(xv, [xb + 1])
            x2 = plsc.load_gather(xv, [xb + 2])
            x3 = plsc.load_gather(xv, [xb + 3])
            sl = pl.ds(k * L, L)
            s0 = x0 * cs[0] + x1 * cs[1] + x2 * cs[2] + x3 * cs[3]
            s1 = x0 * cs[4] + x1 * cs[5] + x2 * cs[6] + x3 * cs[7]
            d0 = x0 * cs[8] + x1 * cs[9] + x2 * cs[10] + x3 * cs[11]
            d1 = x0 * cs[12] + x1 * cs[13] + x2 * cs[14] + x3 * cs[15]
            tv.at[0][sl] = pack_bf16(s0, s1)
            tv.at[1][sl] = pack_bf16(d0, d1)
            return carry2

        pltpu.sync_copy(xf.at[pl.ds(cbase * 4, BCH * 4)], xv)
        lax.fori_loop(0, BCH // L, build, 0)
        pltpu.sync_copy(tv.at[0], sh_s.at[pl.ds(cbase, BCH)])
        pltpu.sync_copy(tv.at[1], sh_d.at[pl.ds(cbase, BCH)])
        return carry

    nch = NBCH // NS + jnp.where(sid < BCH_EXTRA, 1, 0)
    lax.fori_loop(0, nch, build_chunk, 0)
    plsc.subcore_barrier()

    w0v = splat(8)
    w1v = splat(17)
    b0v = splat(18)
    b1v = splat(19)
    gsem = [gsem0, gsem1]
    osem = [osem0, osem1]
    tail = wid < EXTRA

    def idx_load(p, i):
        base = pl.multiple_of((wid + NW * i) * WIN, 8)
        pltpu.sync_copy(src_hbm.at[pl.ds(base, WIN)], src_v.at[p])
        pltpu.sync_copy(dst_hbm.at[pl.ds(base, WIN)], dst_v.at[p])
        pltpu.sync_copy(hc_hbm.at[pl.ds(base, WIN)], hc_v.at[p])

    def gathers(p):
        return [
            pltpu.make_async_copy(sh_s.at[src_v.at[p]], gv.at[p, 0], gsem[p]),
            pltpu.make_async_copy(sh_d.at[dst_v.at[p]], gv.at[p, 1], gsem[p]),
        ]

    def out_copy(p, i):
        blk = (wid + NW * i) * (WIN // 128)
        return pltpu.make_async_copy(
            o_v.at[p], out_hbm.at[pl.ds(blk, WIN // 128)], osem[p])

    def compute(p):
        def step(k, carry2):
            sl = pl.ds(k * L, L)
            h = hc_v.at[p][sl].astype(jnp.float32)
            gs = gv.at[p, 0][sl]
            gd = gv.at[p, 1][sl]
            s0 = plsc.bitcast(gs << 16, jnp.float32)
            s1 = plsc.bitcast(gs & himask, jnp.float32)
            d0 = plsc.bitcast(gd << 16, jnp.float32)
            d1 = plsc.bitcast(gd & himask, jnp.float32)
            o0 = s0 + d0 + h * w0v + b0v
            o1 = s1 + d1 + h * w1v + b1v
            blk = k >> 3
            sub = pl.ds((k & 7) * L, L)
            o_v[p, blk, 0, sub] = o0
            o_v[p, blk, 1, sub] = o1
            return carry2

        lax.fori_loop(0, WIN // L, step, 0)

    def run_window(p, i):
        for g in gathers(p):
            g.wait()
        if i >= 2:
            out_copy(p, i - 2).wait()
        compute(p)
        out_copy(p, i).start()

    idx_load(0, 0)
    for g in gathers(0):
        g.start()

    for i in range(NWIN_MAX):
        p = i % 2
        ni = i + 1
        if ni < NWIN_MAX - 1:
            idx_load(ni % 2, ni)
            for g in gathers(ni % 2):
                g.start()
        elif ni == NWIN_MAX - 1:
            @pl.when(tail)
            def _():
                idx_load(ni % 2, ni)
                for g in gathers(ni % 2):
                    g.start()

        if i < NWIN_MAX - 1:
            run_window(p, i)
        else:
            @pl.when(tail)
            def _():
                run_window(p, i)

    @pl.when(tail)
    def _():
        out_copy(1, NWIN_MAX - 1).wait()

    @pl.when(jnp.logical_not(tail))
    def _():
        out_copy(1, NWIN_MAX - 3).wait()

    out_copy(0, NWIN_MAX - 2).wait()


@functools.partial(
    pl.kernel,
    mesh=plsc.VectorSubcoreMesh(core_axis_name="c", subcore_axis_name="s"),
    out_type=jax.ShapeDtypeStruct((N_EVENTS // 128, 2, 128), jnp.float32),
    compiler_params=pltpu.CompilerParams(needs_layout_passes=False,
                                         use_tc_tiling_on_sc=False),
    scratch_types=[
        pltpu.VMEM((BCH * 4,), jnp.float32),
        pltpu.VMEM((2, BCH), jnp.int32),
        pltpu.VMEM((2, WIN), jnp.int32),
        pltpu.VMEM((2, WIN), jnp.int32),
        pltpu.VMEM((2, WIN), jnp.int32),
        pltpu.VMEM((2, 2, WIN), jnp.int32),
        pltpu.VMEM((2, WIN // 128, 2, 128), jnp.float32),
        pltpu.VMEM((2 * L,), jnp.float32),
        pltpu.VMEM_SHARED((N_NODES,), jnp.int32),
        pltpu.VMEM_SHARED((N_NODES,), jnp.int32),
        pltpu.SemaphoreType.DMA,
        pltpu.SemaphoreType.DMA,
        pltpu.SemaphoreType.DMA,
        pltpu.SemaphoreType.DMA,
    ],
)
def _sc_kernel(*refs):
    _sc_body(*refs)


def kernel(x, src_index, dst_index, history_counts, W, b):
    xf = x.reshape(N_NODES * 4)
    wb = jnp.concatenate([W.reshape(18), b,
                          jnp.zeros(2 * L - 20, jnp.float32)])
    o3 = _sc_kernel(xf, src_index, dst_index, history_counts, wb)
    return o3.swapaxes(1, 2).reshape(N_EVENTS, 2)
